# baseline (speedup 1.0000x reference)
import functools

import jax
import jax.numpy as jnp
from jax import lax
from jax.experimental import pallas as pl
from jax.experimental.pallas import tpu as pltpu
from jax.experimental.pallas import tpu_sc as plsc

_K = 16
_R = 256
_T = 5


def _tc_body(x_ref, xq_ref, wz_ref, wy_ref, bias_ref, idx_ref, y_ref, z_ref):
    xall = x_ref[...]
    xq = xq_ref[...]
    n = xall.shape[0]

    prod = lax.dot_general(
        xq, xall, (((1,), (1,)), ((), ())),
        preferred_element_type=jnp.float32)
    sq_q = jnp.sum(xq * xq, axis=1, keepdims=True)
    ones_row = jnp.ones((1, xall.shape[1]), jnp.float32)
    sq_a = lax.dot_general(
        ones_row, xall * xall, (((1,), (1,)), ((), ())),
        preferred_element_type=jnp.float32,
        precision=lax.Precision.HIGHEST)
    dist = -2.0 * prod + sq_q + sq_a

    big = jnp.float32(3.0e38)
    bigi = jnp.int32(1 << 30)
    r = xq.shape[0]
    nl = 128
    nk = n // nl
    lane = lax.broadcasted_iota(jnp.int32, (r, nl), 1)

    d_list = [dist[:, k * nl:(k + 1) * nl] for k in range(nk)]
    vals, colids = [], []
    for t in range(_T):
        v = d_list[0]
        a = jnp.zeros((r, nl), jnp.int32)
        for k in range(1, nk):
            c = d_list[k]
            lt = c < v
            v = jnp.where(lt, c, v)
            a = jnp.where(lt, k, a)
        vals.append(v)
        colids.append(a * nl + lane)
        if t < _T - 1:
            for k in range(nk):
                d_list[k] = jnp.where(a == k, big, d_list[k])
    vv = jnp.concatenate(vals, axis=1)
    cc = jnp.concatenate(colids, axis=1)

    cols = []
    for p in range(_K + 1):
        m = jnp.min(vv, axis=1, keepdims=True)
        cand = jnp.where(vv == m, cc, bigi)
        mc = jnp.min(cand, axis=1, keepdims=True)
        if p > 0:
            cols.append(mc)
        vv = jnp.where(cand == mc, big, vv)
    idx_local = jnp.concatenate(cols, axis=1)

    sent = vv[:, (_T - 1) * nl: _T * nl]
    trig = jnp.max(jnp.where(sent >= big, jnp.int32(1), jnp.int32(0)))

    def _fallback(_):
        iota = lax.broadcasted_iota(jnp.int32, dist.shape, 1)
        dd = dist
        fcols = []
        for p in range(_K + 1):
            pos = jnp.argmin(dd, axis=1).astype(jnp.int32)[:, None]
            if p > 0:
                fcols.append(pos)
            dd = jnp.where(iota == pos, big, dd)
        return jnp.concatenate(fcols, axis=1)

    idx_ref[...] = lax.cond(trig > 0, _fallback, lambda _: idx_local, 0)

    y_ref[...] = lax.dot_general(
        xq, wy_ref[...], (((1,), (0,)), ((), ())),
        preferred_element_type=jnp.float32,
        precision=lax.Precision.HIGHEST)
    z_ref[...] = lax.dot_general(
        xq, wz_ref[...], (((1,), (0,)), ((), ())),
        preferred_element_type=jnp.float32,
        precision=lax.Precision.HIGHEST) + bias_ref[0:1, :]


def _tc_topk(xb, wz, wy, bias):
    n, d = xb.shape
    fout = wz.shape[1]
    grid = (n // _R,)
    return pl.pallas_call(
        _tc_body,
        grid=grid,
        in_specs=[
            pl.BlockSpec((n, d), lambda i: (0, 0)),
            pl.BlockSpec((_R, d), lambda i: (i, 0)),
            pl.BlockSpec((d, fout), lambda i: (0, 0)),
            pl.BlockSpec((d, fout), lambda i: (0, 0)),
            pl.BlockSpec((1, fout), lambda i: (0, 0)),
        ],
        out_specs=[
            pl.BlockSpec((_R, _K), lambda i: (i, 0)),
            pl.BlockSpec((_R, fout), lambda i: (i, 0)),
            pl.BlockSpec((_R, fout), lambda i: (i, 0)),
        ],
        out_shape=[
            jax.ShapeDtypeStruct((n, _K), jnp.int32),
            jax.ShapeDtypeStruct((n, fout), jnp.float32),
            jax.ShapeDtypeStruct((n, fout), jnp.float32),
        ],
    )(xb, xb, wz, wy, bias)


_NC = 2
_NS = 16
_NW = _NC * _NS
_CH = 8


def _make_sc_gather_max(total_q, fout):
    qpw = total_q // _NW
    nch = qpw // _CH
    chk = _CH * _K
    mesh = plsc.VectorSubcoreMesh(core_axis_name="c", subcore_axis_name="s")

    @functools.partial(
        pl.kernel,
        mesh=mesh,
        compiler_params=pltpu.CompilerParams(use_tc_tiling_on_sc=False),
        out_type=jax.ShapeDtypeStruct((total_q, fout), jnp.float32),
        scratch_types=[
            pltpu.VMEM((qpw * _K,), jnp.int32),
            pltpu.VMEM((2, chk, fout), jnp.float32),
            pltpu.VMEM((qpw, fout), jnp.float32),
            pltpu.VMEM((qpw, fout), jnp.float32),
            pltpu.SemaphoreType.DMA,
            pltpu.SemaphoreType.DMA,
        ],
    )
    def sc_kernel(y_hbm, idx_hbm, z_hbm, out_hbm,
                  idx_all, rows_v, z_big, o_big, gsem, lsem):
        wid = lax.axis_index("s") * _NC + lax.axis_index("c")
        base_q = wid * qpw

        idx_cp = pltpu.async_copy(
            idx_hbm.at[pl.ds(base_q * _K, qpw * _K)], idx_all, lsem)
        z_cp = pltpu.async_copy(z_hbm.at[pl.ds(base_q, qpw)], z_big, lsem)

        def start(cc, b):
            pltpu.async_copy(
                y_hbm.at[idx_all.at[pl.ds(cc * chk, chk)]], rows_v.at[b], gsem)

        def wait_gather(b):
            pltpu.make_async_copy(
                y_hbm.at[pl.ds(0, chk)], rows_v.at[b], gsem).wait()

        def compute(cc, b):
            wait_gather(b)
            for q in range(_CH):
                row = cc * _CH + q
                for col in range(fout // 16):
                    s = pl.ds(col * 16, 16)
                    acc = jnp.maximum(rows_v[b, q * _K, s],
                                      rows_v[b, q * _K + 1, s])
                    for j in range(2, _K):
                        acc = jnp.maximum(acc, rows_v[b, q * _K + j, s])
                    o_big[row, s] = acc + z_big[row, s]

        idx_cp.wait()
        start(0, 0)
        start(1, 1)
        z_cp.wait()

        def body(i, carry):
            cc0 = 2 * i
            compute(cc0, 0)
            start(cc0 + 2, 0)
            compute(cc0 + 1, 1)
            start(cc0 + 3, 1)
            return carry

        lax.fori_loop(0, nch // 2 - 1, body, 0)
        compute(nch - 2, 0)
        compute(nch - 1, 1)
        pltpu.sync_copy(o_big, out_hbm.at[pl.ds(base_q, qpw)])

    return sc_kernel


def kernel(x, W, b):
    bsz, n, d = x.shape
    fout = W.shape[0]
    w1 = W[:, :d]
    w2 = W[:, d:]
    wz = (w1 - w2).T
    wy = w2.T
    bias = b.reshape(1, fout)

    sc_call = _make_sc_gather_max(n, fout)
    outs = []
    for bi in range(bsz):
        idx, y, z = _tc_topk(x[bi], wz, wy, bias)
        outs.append(sc_call(y, idx.reshape(n * _K), z))
    return jnp.stack(outs, axis=0)


# speedup vs baseline: 1.1547x; 1.1547x over previous
import functools

import jax
import jax.numpy as jnp
from jax import lax
from jax.experimental import pallas as pl
from jax.experimental.pallas import tpu as pltpu
from jax.experimental.pallas import tpu_sc as plsc

_K = 16
_R = 256
_T = 5


def _tc_body(x_ref, xq_ref, wz_ref, wy_ref, bias_ref, idx_ref, y_ref, z_ref):
    b = pl.program_id(0)
    xall = x_ref[0]
    xq = xq_ref[0]
    n = xall.shape[0]

    prod = lax.dot_general(
        xq, xall, (((1,), (1,)), ((), ())),
        preferred_element_type=jnp.float32)
    sq_q = jnp.sum(xq * xq, axis=1, keepdims=True)
    ones_row = jnp.ones((1, xall.shape[1]), jnp.float32)
    sq_a = lax.dot_general(
        ones_row, xall * xall, (((1,), (1,)), ((), ())),
        preferred_element_type=jnp.float32,
        precision=lax.Precision.HIGHEST)
    dist = -2.0 * prod + sq_q + sq_a

    big = jnp.float32(3.0e38)
    bigc = jnp.float32(1.0e9)
    r = xq.shape[0]
    nl = 128
    nk = n // nl
    lane = lax.broadcasted_iota(jnp.int32, (r, nl), 1).astype(jnp.float32)

    d_list = [dist[:, k * nl:(k + 1) * nl] for k in range(nk)]
    vals, colids = [], []
    for t in range(_T):
        v = d_list[0]
        a = jnp.zeros((r, nl), jnp.float32)
        for k in range(1, nk):
            c = d_list[k]
            lt = c < v
            v = jnp.where(lt, c, v)
            a = jnp.where(lt, jnp.float32(k), a)
        vals.append(v)
        colids.append(a * jnp.float32(nl) + lane)
        if t < _T - 1:
            for k in range(nk):
                d_list[k] = jnp.where(a == jnp.float32(k), big, d_list[k])
    vv = jnp.concatenate(vals, axis=1)
    cc = jnp.concatenate(colids, axis=1)

    cols = []
    for p in range(_K + 1):
        m = jnp.min(vv, axis=1, keepdims=True)
        cand = jnp.where(vv == m, cc, bigc)
        mc = jnp.min(cand, axis=1, keepdims=True)
        if p > 0:
            cols.append(mc)
        vv = jnp.where(cand == mc, big, vv)
    idx_local = jnp.concatenate(cols, axis=1).astype(jnp.int32)

    sent = vv[:, (_T - 1) * nl: _T * nl]
    trig = jnp.max(jnp.where(sent >= big, jnp.int32(1), jnp.int32(0)))

    def _fallback(_):
        iota = lax.broadcasted_iota(jnp.int32, dist.shape, 1)
        dd = dist
        fcols = []
        for p in range(_K + 1):
            pos = jnp.argmin(dd, axis=1).astype(jnp.int32)[:, None]
            if p > 0:
                fcols.append(pos)
            dd = jnp.where(iota == pos, big, dd)
        return jnp.concatenate(fcols, axis=1)

    idx_local = lax.cond(trig > 0, _fallback, lambda _: idx_local, 0)
    idx_ref[0] = idx_local + b * n

    y_ref[0] = lax.dot_general(
        xq, wy_ref[...], (((1,), (0,)), ((), ())),
        preferred_element_type=jnp.float32,
        precision=lax.Precision.HIGHEST)
    z_ref[0] = lax.dot_general(
        xq, wz_ref[...], (((1,), (0,)), ((), ())),
        preferred_element_type=jnp.float32,
        precision=lax.Precision.HIGHEST) + bias_ref[0:1, :]


def _tc_topk(x, wz, wy, bias):
    bsz, n, d = x.shape
    fout = wz.shape[1]
    grid = (bsz, n // _R)
    return pl.pallas_call(
        _tc_body,
        grid=grid,
        in_specs=[
            pl.BlockSpec((1, n, d), lambda b, i: (b, 0, 0)),
            pl.BlockSpec((1, _R, d), lambda b, i: (b, i, 0)),
            pl.BlockSpec((d, fout), lambda b, i: (0, 0)),
            pl.BlockSpec((d, fout), lambda b, i: (0, 0)),
            pl.BlockSpec((1, fout), lambda b, i: (0, 0)),
        ],
        out_specs=[
            pl.BlockSpec((1, _R, _K), lambda b, i: (b, i, 0)),
            pl.BlockSpec((1, _R, fout), lambda b, i: (b, i, 0)),
            pl.BlockSpec((1, _R, fout), lambda b, i: (b, i, 0)),
        ],
        out_shape=[
            jax.ShapeDtypeStruct((bsz, n, _K), jnp.int32),
            jax.ShapeDtypeStruct((bsz, n, fout), jnp.float32),
            jax.ShapeDtypeStruct((bsz, n, fout), jnp.float32),
        ],
    )(x, x, wz, wy, bias)


_NC = 2
_NS = 16
_NW = _NC * _NS
_CH = 8


def _make_sc_gather_max(total_q, fout):
    qpw = total_q // _NW
    nch = qpw // _CH
    chk = _CH * _K
    mesh = plsc.VectorSubcoreMesh(core_axis_name="c", subcore_axis_name="s")

    @functools.partial(
        pl.kernel,
        mesh=mesh,
        compiler_params=pltpu.CompilerParams(use_tc_tiling_on_sc=False),
        out_type=jax.ShapeDtypeStruct((total_q, fout), jnp.float32),
        scratch_types=[
            pltpu.VMEM((qpw * _K,), jnp.int32),
            pltpu.VMEM((2, chk, fout), jnp.float32),
            pltpu.VMEM((qpw, fout), jnp.float32),
            pltpu.VMEM((qpw, fout), jnp.float32),
            pltpu.SemaphoreType.DMA,
            pltpu.SemaphoreType.DMA,
        ],
    )
    def sc_kernel(y_hbm, idx_hbm, z_hbm, out_hbm,
                  idx_all, rows_v, z_big, o_big, gsem, lsem):
        wid = lax.axis_index("s") * _NC + lax.axis_index("c")
        base_q = wid * qpw

        idx_cp = pltpu.async_copy(
            idx_hbm.at[pl.ds(base_q * _K, qpw * _K)], idx_all, lsem)
        z_cp = pltpu.async_copy(z_hbm.at[pl.ds(base_q, qpw)], z_big, lsem)

        def start(cc, b):
            pltpu.async_copy(
                y_hbm.at[idx_all.at[pl.ds(cc * chk, chk)]], rows_v.at[b], gsem)

        def wait_gather(b):
            pltpu.make_async_copy(
                y_hbm.at[pl.ds(0, chk)], rows_v.at[b], gsem).wait()

        def compute(cc, b):
            wait_gather(b)
            for q in range(_CH):
                row = cc * _CH + q
                for col in range(fout // 16):
                    s = pl.ds(col * 16, 16)
                    acc = jnp.maximum(rows_v[b, q * _K, s],
                                      rows_v[b, q * _K + 1, s])
                    for j in range(2, _K):
                        acc = jnp.maximum(acc, rows_v[b, q * _K + j, s])
                    o_big[row, s] = acc + z_big[row, s]

        idx_cp.wait()
        start(0, 0)
        start(1, 1)
        z_cp.wait()

        def body(i, carry):
            cc0 = 2 * i
            compute(cc0, 0)
            start(cc0 + 2, 0)
            compute(cc0 + 1, 1)
            start(cc0 + 3, 1)
            return carry

        lax.fori_loop(0, nch // 2 - 1, body, 0)
        compute(nch - 2, 0)
        compute(nch - 1, 1)
        pltpu.sync_copy(o_big, out_hbm.at[pl.ds(base_q, qpw)])

    return sc_kernel


def kernel(x, W, b):
    bsz, n, d = x.shape
    fout = W.shape[0]
    w1 = W[:, :d]
    w2 = W[:, d:]
    wz = (w1 - w2).T
    wy = w2.T
    bias = b.reshape(1, fout)

    idx, y, z = _tc_topk(x, wz, wy, bias)

    total_q = bsz * n
    idx_f = idx.reshape(total_q * _K)
    y_f = y.reshape(total_q, fout)
    z_f = z.reshape(total_q, fout)
    out = _make_sc_gather_max(total_q, fout)(y_f, idx_f, z_f)
    return out.reshape(bsz, n, fout)


# speedup vs baseline: 1.2039x; 1.0426x over previous
import functools

import jax
import jax.numpy as jnp
from jax import lax
from jax.experimental import pallas as pl
from jax.experimental.pallas import tpu as pltpu
from jax.experimental.pallas import tpu_sc as plsc

_K = 16
_R = 512
_T = 5


def _tc_body(x_ref, xq_ref, wz_ref, wy_ref, bias_ref, idx_ref, y_ref, z_ref):
    b = pl.program_id(0)
    xall = x_ref[0]
    xq = xq_ref[0]
    n = xall.shape[0]

    prod = lax.dot_general(
        xq, xall, (((1,), (1,)), ((), ())),
        preferred_element_type=jnp.float32)
    sq_q = jnp.sum(xq * xq, axis=1, keepdims=True)
    ones_row = jnp.ones((1, xall.shape[1]), jnp.float32)
    sq_a = lax.dot_general(
        ones_row, xall * xall, (((1,), (1,)), ((), ())),
        preferred_element_type=jnp.float32,
        precision=lax.Precision.HIGHEST)
    dist = -2.0 * prod + sq_q + sq_a

    big = jnp.float32(3.0e38)
    bigc = jnp.float32(1.0e9)
    r = xq.shape[0]
    nl = 128
    nk = n // nl
    lane = lax.broadcasted_iota(jnp.int32, (r, nl), 1).astype(jnp.float32)

    d_list = [dist[:, k * nl:(k + 1) * nl] for k in range(nk)]
    vals, colids = [], []
    for t in range(_T):
        v = d_list[0]
        a = jnp.zeros((r, nl), jnp.float32)
        for k in range(1, nk):
            c = d_list[k]
            lt = c < v
            v = jnp.where(lt, c, v)
            a = jnp.where(lt, jnp.float32(k), a)
        vals.append(v)
        colids.append(a * jnp.float32(nl) + lane)
        if t < _T - 1:
            for k in range(nk):
                d_list[k] = jnp.where(a == jnp.float32(k), big, d_list[k])
    vv = jnp.concatenate(vals, axis=1)
    cc = jnp.concatenate(colids, axis=1)

    cols = []
    for p in range(_K + 1):
        m = jnp.min(vv, axis=1, keepdims=True)
        cand = jnp.where(vv == m, cc, bigc)
        mc = jnp.min(cand, axis=1, keepdims=True)
        if p > 0:
            cols.append(mc)
        vv = jnp.where(cand == mc, big, vv)
    idx_local = jnp.concatenate(cols, axis=1).astype(jnp.int32)

    sent = vv[:, (_T - 1) * nl: _T * nl]
    trig = jnp.max(jnp.where(sent >= big, jnp.int32(1), jnp.int32(0)))

    def _fallback(_):
        iota = lax.broadcasted_iota(jnp.int32, dist.shape, 1)
        dd = dist
        fcols = []
        for p in range(_K + 1):
            pos = jnp.argmin(dd, axis=1).astype(jnp.int32)[:, None]
            if p > 0:
                fcols.append(pos)
            dd = jnp.where(iota == pos, big, dd)
        return jnp.concatenate(fcols, axis=1)

    idx_local = lax.cond(trig > 0, _fallback, lambda _: idx_local, 0)
    idx_ref[0] = idx_local + b * n

    y_ref[0] = lax.dot_general(
        xq, wy_ref[...], (((1,), (0,)), ((), ())),
        preferred_element_type=jnp.float32,
        precision=lax.Precision.HIGHEST)
    z_ref[0] = lax.dot_general(
        xq, wz_ref[...], (((1,), (0,)), ((), ())),
        preferred_element_type=jnp.float32,
        precision=lax.Precision.HIGHEST) + bias_ref[0:1, :]


def _tc_topk(x, wz, wy, bias):
    bsz, n, d = x.shape
    fout = wz.shape[1]
    grid = (bsz, n // _R)
    return pl.pallas_call(
        _tc_body,
        grid=grid,
        in_specs=[
            pl.BlockSpec((1, n, d), lambda b, i: (b, 0, 0)),
            pl.BlockSpec((1, _R, d), lambda b, i: (b, i, 0)),
            pl.BlockSpec((d, fout), lambda b, i: (0, 0)),
            pl.BlockSpec((d, fout), lambda b, i: (0, 0)),
            pl.BlockSpec((1, fout), lambda b, i: (0, 0)),
        ],
        out_specs=[
            pl.BlockSpec((1, _R, _K), lambda b, i: (b, i, 0)),
            pl.BlockSpec((1, _R, fout), lambda b, i: (b, i, 0)),
            pl.BlockSpec((1, _R, fout), lambda b, i: (b, i, 0)),
        ],
        out_shape=[
            jax.ShapeDtypeStruct((bsz, n, _K), jnp.int32),
            jax.ShapeDtypeStruct((bsz, n, fout), jnp.float32),
            jax.ShapeDtypeStruct((bsz, n, fout), jnp.float32),
        ],
    )(x, x, wz, wy, bias)


_NC = 2
_NS = 16
_NW = _NC * _NS
_CH = 8


def _make_sc_gather_max(total_q, fout):
    qpw = total_q // _NW
    nch = qpw // _CH
    chk = _CH * _K
    mesh = plsc.VectorSubcoreMesh(core_axis_name="c", subcore_axis_name="s")

    @functools.partial(
        pl.kernel,
        mesh=mesh,
        compiler_params=pltpu.CompilerParams(use_tc_tiling_on_sc=False),
        out_type=jax.ShapeDtypeStruct((total_q, fout), jnp.float32),
        scratch_types=[
            pltpu.VMEM((qpw * _K,), jnp.int32),
            pltpu.VMEM((2, chk, fout), jnp.float32),
            pltpu.VMEM((qpw, fout), jnp.float32),
            pltpu.VMEM((qpw, fout), jnp.float32),
            pltpu.SemaphoreType.DMA,
            pltpu.SemaphoreType.DMA,
        ],
    )
    def sc_kernel(y_hbm, idx_hbm, z_hbm, out_hbm,
                  idx_all, rows_v, z_big, o_big, gsem, lsem):
        wid = lax.axis_index("s") * _NC + lax.axis_index("c")
        base_q = wid * qpw

        idx_cp = pltpu.async_copy(
            idx_hbm.at[pl.ds(base_q * _K, qpw * _K)], idx_all, lsem)
        z_cp = pltpu.async_copy(z_hbm.at[pl.ds(base_q, qpw)], z_big, lsem)

        def start(cc, b):
            pltpu.async_copy(
                y_hbm.at[idx_all.at[pl.ds(cc * chk, chk)]], rows_v.at[b], gsem)

        def wait_gather(b):
            pltpu.make_async_copy(
                y_hbm.at[pl.ds(0, chk)], rows_v.at[b], gsem).wait()

        def compute(cc, b):
            wait_gather(b)
            for q in range(_CH):
                row = cc * _CH + q
                for col in range(fout // 16):
                    s = pl.ds(col * 16, 16)
                    acc = jnp.maximum(rows_v[b, q * _K, s],
                                      rows_v[b, q * _K + 1, s])
                    for j in range(2, _K):
                        acc = jnp.maximum(acc, rows_v[b, q * _K + j, s])
                    o_big[row, s] = acc + z_big[row, s]

        idx_cp.wait()
        start(0, 0)
        start(1, 1)
        z_cp.wait()

        def body(i, carry):
            cc0 = 2 * i
            compute(cc0, 0)
            start(cc0 + 2, 0)
            compute(cc0 + 1, 1)
            start(cc0 + 3, 1)
            return carry

        lax.fori_loop(0, nch // 2 - 1, body, 0)
        compute(nch - 2, 0)
        compute(nch - 1, 1)
        pltpu.sync_copy(o_big, out_hbm.at[pl.ds(base_q, qpw)])

    return sc_kernel


def kernel(x, W, b):
    bsz, n, d = x.shape
    fout = W.shape[0]
    w1 = W[:, :d]
    w2 = W[:, d:]
    wz = (w1 - w2).T
    wy = w2.T
    bias = b.reshape(1, fout)

    idx, y, z = _tc_topk(x, wz, wy, bias)

    total_q = bsz * n
    idx_f = idx.reshape(total_q * _K)
    y_f = y.reshape(total_q, fout)
    z_f = z.reshape(total_q, fout)
    out = _make_sc_gather_max(total_q, fout)(y_f, idx_f, z_f)
    return out.reshape(bsz, n, fout)
